# baseline (device time: 24030 ns/iter reference)
import os

import jax
import jax.numpy as jnp
from jax import lax
from jax.experimental import pallas as pl
from jax.experimental.pallas import tpu as pltpu

try:
    _PROBE = open(os.path.join(os.path.dirname(__file__), "kprobe.txt")).read().strip()
except OSError:
    _PROBE = ""

N_DEV = 4
B_LOC = 2
SQ = 128
SKV = 128
HQ = 16
H_CHUNK = 4
DH = 64
D_MODEL = 512
HD = H_CHUNK * DH


_BLK = 64
_NBLK = SQ // _BLK


def _chunk_ctx(g, x2, k_ref, v_ref, wq_buf, scales, s, mask):
    q = jnp.dot(x2, wq_buf[s].astype(jnp.bfloat16),
                preferred_element_type=jnp.float32)
    q = (q * (scales[s, 0, 0] * 0.125)).astype(jnp.bfloat16)
    rows = []
    for b in range(B_LOC):
        kc = k_ref[b, :, pl.ds(g * HD, HD)]
        vc = v_ref[b, :, pl.ds(g * HD, HD)]
        parts = []
        for j in range(H_CHUNK):
            qh = q[b * SQ:(b + 1) * SQ, j * DH:(j + 1) * DH]
            kh = kc[:, j * DH:(j + 1) * DH]
            vh = vc[:, j * DH:(j + 1) * DH]
            blocks = []
            for t in range(_NBLK):
                r = slice(t * _BLK, (t + 1) * _BLK)
                sc = lax.dot_general(
                    qh[r], kh[r], (((1,), (1,)), ((), ())),
                    preferred_element_type=jnp.float32)
                w = jnp.exp(sc)
                w = w / jnp.sum(w, axis=-1, keepdims=True)
                blocks.append(jnp.dot(w.astype(jnp.bfloat16), vh[r],
                                      preferred_element_type=jnp.float32))
            parts.append(jnp.concatenate(blocks, axis=0))
        rows.append(jnp.concatenate(parts, axis=1))
    return jnp.concatenate(rows, axis=0).astype(jnp.bfloat16)


def _chunk_compute(my, s, x2, k_ref, v_ref, wq_buf, wo_buf, scales, mask):
    g = lax.rem(my + s, N_DEV)
    ctx = _chunk_ctx(g, x2, k_ref, v_ref, wq_buf, scales, s, mask)
    out = jnp.dot(ctx, wo_buf[s].astype(jnp.bfloat16),
                  preferred_element_type=jnp.float32)
    return out * scales[s, 0, 1]


def _mask():
    row = lax.broadcasted_iota(jnp.int32, (SQ, SKV), 0) // 64
    col = lax.broadcasted_iota(jnp.int32, (SQ, SKV), 1) // 64
    return (row == col) | ((col % 4) == (row % 4))


def _quantize(w):
    scale = jnp.max(jnp.abs(w)) / 127.0
    return jnp.round(w / scale).astype(jnp.int8), scale


def _scale_tile(swq, swo):
    lane = lax.broadcasted_iota(jnp.int32, (8, 128), 1)
    return jnp.where(lane == 0, swq, jnp.where(lane == 1, swo, 0.0))


def _body(x_ref, wq_ref, k_ref, v_ref, wo_ref, out_ref,
          wq_buf, wo_buf, scales, send_sems, recv_sems):
    my = lax.axis_index("i")
    left = lax.rem(my + N_DEV - 1, N_DEV)
    right = lax.rem(my + 1, N_DEV)

    barrier = pltpu.get_barrier_semaphore()
    pl.semaphore_signal(barrier, inc=1, device_id=(left,),
                        device_id_type=pl.DeviceIdType.MESH)
    pl.semaphore_signal(barrier, inc=1, device_id=(right,),
                        device_id_type=pl.DeviceIdType.MESH)
    pl.semaphore_wait(barrier, 2)

    wq_i8, swq = _quantize(wq_ref[:, :])
    wo_i8, swo = _quantize(wo_ref[:, :])
    wq_buf[0, :, :] = wq_i8
    wo_buf[0, :, :] = wo_i8
    scales[0, :, :] = _scale_tile(swq, swo)

    if _PROBE == "compute":
        for s in range(1, N_DEV):
            wq_buf[s, :, :] = wq_i8
            wo_buf[s, :, :] = wo_i8
            scales[s, :, :] = _scale_tile(swq, swo)
        x2 = x_ref[:, :, :].reshape(B_LOC * SQ, D_MODEL).astype(jnp.bfloat16)
        mask = None
        acc = _chunk_compute(my, 0, x2, k_ref, v_ref, wq_buf, wo_buf, scales, mask)
        for s in range(1, N_DEV):
            acc = acc + _chunk_compute(my, s, x2, k_ref, v_ref,
                                       wq_buf, wo_buf, scales, mask)
        out_ref[:, :, :] = acc.reshape(B_LOC, SQ, D_MODEL)
        return

    def copy(src_slot, dst_slot, buf, sem, dev):
        return pltpu.make_async_remote_copy(
            src_ref=buf.at[src_slot], dst_ref=buf.at[dst_slot],
            send_sem=send_sems.at[sem], recv_sem=recv_sems.at[sem],
            device_id=(dev,), device_id_type=pl.DeviceIdType.MESH)

    e1 = copy(0, 3, scales, 6, right)
    e3 = copy(0, 1, scales, 7, left)
    d1 = copy(0, 3, wq_buf, 0, right)
    d3 = copy(0, 1, wq_buf, 2, left)
    d2 = copy(0, 3, wo_buf, 1, right)
    d4 = copy(0, 1, wo_buf, 3, left)
    e1.start()
    e3.start()
    d1.start()
    d3.start()
    d2.start()
    d4.start()

    if _PROBE == "comm":
        d1.wait()
        e1.wait()
        d5 = copy(3, 2, wq_buf, 4, right)
        e5 = copy(3, 2, scales, 8, right)
        d5.start()
        e5.start()
        d4.wait()
        d6 = copy(1, 2, wo_buf, 5, left)
        d6.start()
        e3.wait()
        d2.wait()
        d3.wait()
        d5.wait()
        d6.wait()
        e5.wait()
        out_ref[:, :, :] = jnp.zeros((B_LOC, SQ, D_MODEL), jnp.float32)
        return

    mask = None
    x2 = x_ref[:, :, :].reshape(B_LOC * SQ, D_MODEL).astype(jnp.bfloat16)
    acc = _chunk_compute(my, 0, x2, k_ref, v_ref, wq_buf, wo_buf, scales, mask)

    d1.wait()
    e1.wait()
    d5 = copy(3, 2, wq_buf, 4, right)
    e5 = copy(3, 2, scales, 8, right)
    d5.start()
    e5.start()
    d4.wait()
    d6 = copy(1, 2, wo_buf, 5, left)
    d6.start()

    d2.wait()
    acc = acc + _chunk_compute(my, 3, x2, k_ref, v_ref, wq_buf, wo_buf,
                               scales, mask)
    e3.wait()
    d3.wait()
    acc = acc + _chunk_compute(my, 1, x2, k_ref, v_ref, wq_buf, wo_buf,
                               scales, mask)
    d5.wait()
    e5.wait()
    ctx2 = _chunk_ctx(lax.rem(my + 2, N_DEV), x2, k_ref, v_ref, wq_buf,
                      scales, 2, mask)
    d6.wait()
    acc = acc + jnp.dot(ctx2, wo_buf[2].astype(jnp.bfloat16),
                        preferred_element_type=jnp.float32) * scales[2, 0, 1]

    out_ref[:, :, :] = acc.reshape(B_LOC, SQ, D_MODEL)


def kernel(x, Wq, K_ext, V_ext, Wo):
    my = lax.axis_index("i")

    def prep(a):
        a = lax.dynamic_slice_in_dim(a, my * B_LOC, B_LOC, axis=0)
        return a.reshape(B_LOC, SKV, HQ * DH).astype(jnp.bfloat16)

    return pl.pallas_call(
        _body,
        out_shape=jax.ShapeDtypeStruct((B_LOC, SQ, D_MODEL), jnp.float32),
        in_specs=[pl.BlockSpec(memory_space=pltpu.VMEM)] * 5,
        out_specs=pl.BlockSpec(memory_space=pltpu.VMEM),
        scratch_shapes=[
            pltpu.VMEM((N_DEV, D_MODEL, HD), jnp.int8),
            pltpu.VMEM((N_DEV, HD, D_MODEL), jnp.int8),
            pltpu.VMEM((N_DEV, 8, 128), jnp.float32),
            pltpu.SemaphoreType.DMA((9,)),
            pltpu.SemaphoreType.DMA((9,)),
        ],
        compiler_params=pltpu.CompilerParams(collective_id=0),
    )(x, Wq, prep(K_ext), prep(V_ext), Wo)


# device time: 18536 ns/iter; 1.2964x vs baseline; 1.2964x over previous
import os

import jax
import jax.numpy as jnp
from jax import lax
from jax.experimental import pallas as pl
from jax.experimental.pallas import tpu as pltpu

try:
    _PROBE = open(os.path.join(os.path.dirname(__file__), "kprobe.txt")).read().strip()
except OSError:
    _PROBE = ""

N_DEV = 4
B_LOC = 2
SQ = 128
SKV = 128
HQ = 16
H_CHUNK = 4
DH = 64
D_MODEL = 512
HD = H_CHUNK * DH


def _chunk_ctx(g, x2, k_ref, v_ref, wq_buf, scales, s, mask):
    q = jnp.dot(x2, wq_buf[s].astype(jnp.bfloat16),
                preferred_element_type=jnp.float32)
    q = (q * (scales[s, 0, 0] * 0.125)).astype(jnp.bfloat16)
    rows = []
    for b in range(B_LOC):
        kc = k_ref[b, :, pl.ds(g * HD, HD)]
        vc = v_ref[b, :, pl.ds(g * HD, HD)]
        parts = []
        for j in range(H_CHUNK):
            qh = q[b * SQ:(b + 1) * SQ, j * DH:(j + 1) * DH]
            sc = lax.dot_general(
                qh, kc[:, j * DH:(j + 1) * DH], (((1,), (1,)), ((), ())),
                preferred_element_type=jnp.float32)
            w = jnp.exp(jnp.where(mask, sc, -1e9))
            w = w / jnp.sum(w, axis=-1, keepdims=True)
            parts.append(jnp.dot(w.astype(jnp.bfloat16),
                                 vc[:, j * DH:(j + 1) * DH],
                                 preferred_element_type=jnp.float32))
        rows.append(jnp.concatenate(parts, axis=1))
    return jnp.concatenate(rows, axis=0).astype(jnp.bfloat16)


def _chunk_compute(my, s, x2, k_ref, v_ref, wq_buf, wo_buf, scales, mask):
    g = lax.rem(my + s, N_DEV)
    ctx = _chunk_ctx(g, x2, k_ref, v_ref, wq_buf, scales, s, mask)
    out = jnp.dot(ctx, wo_buf[s].astype(jnp.bfloat16),
                  preferred_element_type=jnp.float32)
    return out * scales[s, 0, 1]


def _mask():
    row = lax.broadcasted_iota(jnp.int32, (SQ, SKV), 0) // 64
    col = lax.broadcasted_iota(jnp.int32, (SQ, SKV), 1) // 64
    return (row == col) | ((col % 4) == (row % 4))


def _quantize(w):
    scale = jnp.max(jnp.abs(w)) / 127.0
    return jnp.round(w / scale).astype(jnp.int8), scale


def _scale_tile(swq, swo):
    lane = lax.broadcasted_iota(jnp.int32, (8, 128), 1)
    return jnp.where(lane == 0, swq, jnp.where(lane == 1, swo, 0.0))


def _body(x_ref, wq_ref, k_ref, v_ref, wo_ref, out_ref,
          wq_buf, wo_buf, scales, send_sems, recv_sems):
    my = lax.axis_index("i")
    left = lax.rem(my + N_DEV - 1, N_DEV)
    right = lax.rem(my + 1, N_DEV)

    barrier = pltpu.get_barrier_semaphore()
    pl.semaphore_signal(barrier, inc=1, device_id=(left,),
                        device_id_type=pl.DeviceIdType.MESH)
    pl.semaphore_signal(barrier, inc=1, device_id=(right,),
                        device_id_type=pl.DeviceIdType.MESH)
    pl.semaphore_wait(barrier, 2)

    wq_i8, swq = _quantize(wq_ref[:, :])
    wq_buf[0, :, :] = wq_i8

    if _PROBE == "compute":
        wo_i8, swo = _quantize(wo_ref[:, :])
        wo_buf[0, :, :] = wo_i8
        scales[0, :, :] = _scale_tile(swq, swo)
        for s in range(1, N_DEV):
            wq_buf[s, :, :] = wq_i8
            wo_buf[s, :, :] = wo_i8
            scales[s, :, :] = _scale_tile(swq, swo)
        x2 = x_ref[:, :, :].reshape(B_LOC * SQ, D_MODEL).astype(jnp.bfloat16)
        mask = _mask()
        acc = _chunk_compute(my, 0, x2, k_ref, v_ref, wq_buf, wo_buf, scales, mask)
        for s in range(1, N_DEV):
            acc = acc + _chunk_compute(my, s, x2, k_ref, v_ref,
                                       wq_buf, wo_buf, scales, mask)
        out_ref[:, :, :] = acc.reshape(B_LOC, SQ, D_MODEL)
        return

    def copy(src_slot, dst_slot, buf, sem, dev):
        return pltpu.make_async_remote_copy(
            src_ref=buf.at[src_slot], dst_ref=buf.at[dst_slot],
            send_sem=send_sems.at[sem], recv_sem=recv_sems.at[sem],
            device_id=(dev,), device_id_type=pl.DeviceIdType.MESH)

    d1 = copy(0, 3, wq_buf, 0, right)
    d3 = copy(0, 1, wq_buf, 2, left)
    d1.start()
    d3.start()

    wo_i8, swo = _quantize(wo_ref[:, :])
    wo_buf[0, :, :] = wo_i8
    d2 = copy(0, 3, wo_buf, 1, right)
    d4 = copy(0, 1, wo_buf, 3, left)
    d2.start()
    d4.start()

    scales[0, :, :] = _scale_tile(swq, swo)
    e1 = copy(0, 3, scales, 6, right)
    e3 = copy(0, 1, scales, 7, left)
    e1.start()
    e3.start()

    if _PROBE == "comm":
        d1.wait()
        e1.wait()
        d5 = copy(3, 2, wq_buf, 4, right)
        e5 = copy(3, 2, scales, 8, right)
        d5.start()
        e5.start()
        d4.wait()
        d6 = copy(1, 2, wo_buf, 5, left)
        d6.start()
        e3.wait()
        d2.wait()
        d3.wait()
        d5.wait()
        d6.wait()
        e5.wait()
        out_ref[:, :, :] = jnp.zeros((B_LOC, SQ, D_MODEL), jnp.float32)
        return

    mask = _mask()
    x2 = x_ref[:, :, :].reshape(B_LOC * SQ, D_MODEL).astype(jnp.bfloat16)
    acc = _chunk_compute(my, 0, x2, k_ref, v_ref, wq_buf, wo_buf, scales, mask)

    d1.wait()
    e1.wait()
    d5 = copy(3, 2, wq_buf, 4, right)
    e5 = copy(3, 2, scales, 8, right)
    d5.start()
    e5.start()
    d4.wait()
    d6 = copy(1, 2, wo_buf, 5, left)
    d6.start()

    d2.wait()
    acc = acc + _chunk_compute(my, 3, x2, k_ref, v_ref, wq_buf, wo_buf,
                               scales, mask)
    e3.wait()
    d3.wait()
    acc = acc + _chunk_compute(my, 1, x2, k_ref, v_ref, wq_buf, wo_buf,
                               scales, mask)
    d5.wait()
    e5.wait()
    ctx2 = _chunk_ctx(lax.rem(my + 2, N_DEV), x2, k_ref, v_ref, wq_buf,
                      scales, 2, mask)
    d6.wait()
    acc = acc + jnp.dot(ctx2, wo_buf[2].astype(jnp.bfloat16),
                        preferred_element_type=jnp.float32) * scales[2, 0, 1]

    out_ref[:, :, :] = acc.reshape(B_LOC, SQ, D_MODEL)


def kernel(x, Wq, K_ext, V_ext, Wo):
    my = lax.axis_index("i")

    def prep(a):
        a = lax.dynamic_slice_in_dim(a, my * B_LOC, B_LOC, axis=0)
        return a.reshape(B_LOC, SKV, HQ * DH).astype(jnp.bfloat16)

    return pl.pallas_call(
        _body,
        out_shape=jax.ShapeDtypeStruct((B_LOC, SQ, D_MODEL), jnp.float32),
        in_specs=[pl.BlockSpec(memory_space=pltpu.VMEM)] * 5,
        out_specs=pl.BlockSpec(memory_space=pltpu.VMEM),
        scratch_shapes=[
            pltpu.VMEM((N_DEV, D_MODEL, HD), jnp.int8),
            pltpu.VMEM((N_DEV, HD, D_MODEL), jnp.int8),
            pltpu.VMEM((N_DEV, 8, 128), jnp.float32),
            pltpu.SemaphoreType.DMA((9,)),
            pltpu.SemaphoreType.DMA((9,)),
        ],
        compiler_params=pltpu.CompilerParams(collective_id=0),
    )(x, Wq, prep(K_ext), prep(V_ext), Wo)
